# baseline (speedup 1.0000x reference)
import functools

import jax
import jax.numpy as jnp
from jax import lax
from jax.experimental import pallas as pl
from jax.experimental.pallas import tpu as pltpu
from jax.experimental.pallas import tpu_sc as plsc

BATCH = 16384
D = 64
HD = D // 2
NROW = 250000
NC, NS = 2, 16
NW = NC * NS
B_PER_W = BATCH // NW
HALF = B_PER_W // 2
L = 16

_SC_MESH = plsc.VectorSubcoreMesh(core_axis_name="c", subcore_axis_name="s")

_ROWS_T = jax.ShapeDtypeStruct((BATCH, 128), jnp.int32)


@functools.partial(
    pl.kernel,
    mesh=_SC_MESH,
    out_type=(_ROWS_T, _ROWS_T, _ROWS_T, _ROWS_T),
    scratch_types=[
        pltpu.VMEM((B_PER_W,), jnp.int32),
        pltpu.VMEM((B_PER_W,), jnp.int32),
        pltpu.VMEM((HALF, 128), jnp.int32),
        pltpu.VMEM((HALF, 128), jnp.int32),
        pltpu.SemaphoreType.DMA,
        pltpu.SemaphoreType.DMA,
        pltpu.SemaphoreType.DMA,
        pltpu.SemaphoreType.DMA,
    ],
)
def _gather4(uid_hbm, iid_hbm, umf_hbm, imf_hbm, uneu_hbm, ineu_hbm,
             out_umf, out_imf, out_uneu, out_ineu,
             idx_u, idx_i, buf_a, buf_b, sem_a, sem_b, sem_wa, sem_wb):
    wid = lax.axis_index("s") * NC + lax.axis_index("c")
    base = wid * B_PER_W
    pltpu.sync_copy(uid_hbm.at[pl.ds(base, B_PER_W)], idx_u)
    pltpu.sync_copy(iid_hbm.at[pl.ds(base, B_PER_W)], idx_i)
    for k in range(B_PER_W // L):
        sl = pl.ds(k * L, L)
        idx_u[sl] = lax.shift_right_logical(idx_u[sl], 2)
        idx_i[sl] = lax.shift_right_logical(idx_i[sl], 2)

    jobs = ((umf_hbm, idx_u, out_umf), (imf_hbm, idx_i, out_imf),
            (uneu_hbm, idx_u, out_uneu), (ineu_hbm, idx_i, out_ineu))
    for tbl, idx, out in jobs:
        g0 = pltpu.async_copy(tbl.at[idx.at[pl.ds(0, HALF)]], buf_a, sem_a)
        g1 = pltpu.async_copy(tbl.at[idx.at[pl.ds(HALF, HALF)]], buf_b, sem_b)
        g0.wait()
        w0 = pltpu.async_copy(buf_a, out.at[pl.ds(base, HALF)], sem_wa)
        g1.wait()
        w1 = pltpu.async_copy(buf_b, out.at[pl.ds(base + HALF, HALF)], sem_wb)
        w0.wait()
        w1.wait()


def _unpack(words):
    lo = lax.bitcast_convert_type(words << 16, jnp.float32)
    hi = lax.bitcast_convert_type(words & jnp.int32(-65536), jnp.float32)
    return jnp.concatenate([lo, hi], axis=1)


def _mlp_body(uid_ref, iid_ref, umf_ref, imf_ref, uneu_ref, ineu_ref,
              w0a_ref, w0b_ref, b0_ref, w1t_ref, b1_ref, out_ref):
    def pick(rows, bits):
        q0 = rows[:, 0 * HD:1 * HD]
        q1 = rows[:, 1 * HD:2 * HD]
        q2 = rows[:, 2 * HD:3 * HD]
        q3 = rows[:, 3 * HD:4 * HD]
        b0 = (bits & 1)[:, :HD] == 1
        b1 = (bits & 2)[:, :HD] == 2
        return _unpack(jnp.where(b1, jnp.where(b0, q3, q2),
                                 jnp.where(b0, q1, q0)))

    ub = uid_ref[...]
    ib = iid_ref[...]
    umf = pick(umf_ref[...], ub)
    imf = pick(imf_ref[...], ib)
    uneu = pick(uneu_ref[...], ub)
    ineu = pick(ineu_ref[...], ib)
    h0 = jax.nn.sigmoid(
        jnp.dot(uneu, w0a_ref[...], preferred_element_type=jnp.float32)
        + jnp.dot(ineu, w0b_ref[...], preferred_element_type=jnp.float32)
        + b0_ref[...])
    h1 = jax.nn.sigmoid(
        jnp.dot(h0, w1t_ref[...], preferred_element_type=jnp.float32)
        + b1_ref[...])
    gmf = jnp.sum(umf * imf, axis=1)
    out_ref[...] = gmf + jnp.sum(h1, axis=1)


_BLK = 2048


def _mlp(uidb, iidb, umf, imf, uneu, ineu, w0a, w0b, b0, w1t, b1):
    grid = (BATCH // _BLK,)
    rows_spec = pl.BlockSpec((_BLK, 128), lambda i: (i, 0))
    full = lambda shape: pl.BlockSpec(shape, lambda i: (0,) * len(shape))
    return pl.pallas_call(
        _mlp_body,
        grid=grid,
        in_specs=[
            rows_spec, rows_spec,
            rows_spec, rows_spec, rows_spec, rows_spec,
            full((D, 128)), full((D, 128)), full((1, 128)),
            full((128, 64)), full((1, 64)),
        ],
        out_specs=pl.BlockSpec((_BLK,), lambda i: (i,)),
        out_shape=jax.ShapeDtypeStruct((BATCH,), jnp.float32),
    )(uidb, iidb, umf, imf, uneu, ineu, w0a, w0b, b0, w1t, b1)


def _to_slices(t):
    tb = t.astype(jnp.bfloat16)
    st = jnp.stack([tb[:, :HD], tb[:, HD:]], axis=-1)
    return lax.bitcast_convert_type(st, jnp.int32).reshape(NROW, 128)


def kernel(user_id, item_id, users_mf, items_mf, users_neu, items_neu,
           W0, b0, W1, b1):
    uid = user_id.astype(jnp.int32)
    iid = item_id.astype(jnp.int32)
    tabs = [_to_slices(t) for t in (users_mf, items_mf, users_neu, items_neu)]
    rows = _gather4(uid, iid, *tabs)
    uidb = jnp.broadcast_to(uid[:, None], (BATCH, 128))
    iidb = jnp.broadcast_to(iid[:, None], (BATCH, 128))
    w0a = W0[:, :D].T
    w0b = W0[:, D:].T
    w1t = W1.T
    return _mlp(uidb, iidb, *rows, w0a, w0b,
                b0.reshape(1, -1), w1t, b1.reshape(1, -1))


# speedup vs baseline: 2.1142x; 2.1142x over previous
import functools

import jax
import jax.numpy as jnp
from jax import lax
from jax.experimental import pallas as pl
from jax.experimental.pallas import tpu as pltpu
from jax.experimental.pallas import tpu_sc as plsc

BATCH = 16384
D = 64
HD = D // 2
N = 1000000
QSH = 18
NROW = 1 << QSH
NC, NS = 2, 16
NW = NC * NS
B_PER_W = BATCH // NW
HALF = B_PER_W // 2
L = 16


_PBLK = 1024
_PGRID = NROW // _PBLK
_NINB = (N + _PBLK - 1) // _PBLK


def _rne_hi16(x):
    xi = lax.bitcast_convert_type(x, jnp.int32)
    r = xi + jnp.int32(0x7FFF) + (lax.shift_right_logical(xi, 16) & 1)
    return lax.shift_right_logical(r, 16)


def _prep_body(*refs):
    in_refs = refs[:16]
    eye_ref = refs[16]
    out_refs = refs[17:21]
    eye = eye_ref[...]
    for t in range(4):
        quarters = []
        for q in range(4):
            x = in_refs[4 * t + q][...]
            xt = lax.dot_general(x, eye, (((0,), (0,)), ((), ())),
                                 preferred_element_type=jnp.float32)
            lo = _rne_hi16(xt[:, :HD])
            hi = _rne_hi16(xt[:, HD:])
            quarters.append(lo | lax.shift_left(hi, 16))
        out_refs[t][...] = jnp.concatenate(quarters, axis=1)


def _prep(tabs_t, eye):
    def in_spec(q):
        base = q * (NROW // _PBLK)
        return pl.BlockSpec(
            (D, _PBLK), lambda i, b=base: (0, jnp.minimum(i + b, _NINB - 1)))

    in_specs = [in_spec(q) for _ in range(4) for q in range(4)]
    in_specs.append(pl.BlockSpec((D, D), lambda i: (0, 0)))
    out_spec = pl.BlockSpec((_PBLK, 128), lambda i: (i, 0))
    out_t = jax.ShapeDtypeStruct((NROW, 128), jnp.int32)
    ins = []
    for t in tabs_t:
        ins.extend([t, t, t, t])
    ins.append(eye)
    return pl.pallas_call(
        _prep_body,
        grid=(_PGRID,),
        in_specs=in_specs,
        out_specs=(out_spec,) * 4,
        out_shape=(out_t,) * 4,
    )(*ins)


_SC_MESH = plsc.VectorSubcoreMesh(core_axis_name="c", subcore_axis_name="s")

_ROWS_T = jax.ShapeDtypeStruct((BATCH, 128), jnp.int32)


@functools.partial(
    pl.kernel,
    mesh=_SC_MESH,
    out_type=(_ROWS_T, _ROWS_T, _ROWS_T, _ROWS_T),
    scratch_types=[
        pltpu.VMEM((B_PER_W,), jnp.int32),
        pltpu.VMEM((B_PER_W,), jnp.int32),
        pltpu.VMEM((HALF, 128), jnp.int32),
        pltpu.VMEM((HALF, 128), jnp.int32),
        pltpu.SemaphoreType.DMA,
        pltpu.SemaphoreType.DMA,
        pltpu.SemaphoreType.DMA,
        pltpu.SemaphoreType.DMA,
    ],
)
def _gather4(uid_hbm, iid_hbm, umf_hbm, imf_hbm, uneu_hbm, ineu_hbm,
             out_umf, out_imf, out_uneu, out_ineu,
             idx_u, idx_i, buf_a, buf_b, sem_a, sem_b, sem_wa, sem_wb):
    wid = lax.axis_index("s") * NC + lax.axis_index("c")
    base = wid * B_PER_W
    pltpu.sync_copy(uid_hbm.at[pl.ds(base, B_PER_W)], idx_u)
    pltpu.sync_copy(iid_hbm.at[pl.ds(base, B_PER_W)], idx_i)
    mask = jnp.int32(NROW - 1)
    for k in range(B_PER_W // L):
        sl = pl.ds(k * L, L)
        idx_u[sl] = idx_u[sl] & mask
        idx_i[sl] = idx_i[sl] & mask

    jobs = ((umf_hbm, idx_u, out_umf), (imf_hbm, idx_i, out_imf),
            (uneu_hbm, idx_u, out_uneu), (ineu_hbm, idx_i, out_ineu))
    for tbl, idx, out in jobs:
        g0 = pltpu.async_copy(tbl.at[idx.at[pl.ds(0, HALF)]], buf_a, sem_a)
        g1 = pltpu.async_copy(tbl.at[idx.at[pl.ds(HALF, HALF)]], buf_b, sem_b)
        g0.wait()
        w0 = pltpu.async_copy(buf_a, out.at[pl.ds(base, HALF)], sem_wa)
        g1.wait()
        w1 = pltpu.async_copy(buf_b, out.at[pl.ds(base + HALF, HALF)], sem_wb)
        w0.wait()
        w1.wait()


def _unpack(words):
    lo = lax.bitcast_convert_type(lax.shift_left(words, 16), jnp.float32)
    hi = lax.bitcast_convert_type(words & jnp.int32(-65536), jnp.float32)
    return jnp.concatenate([lo, hi], axis=1)


def _mlp_body(uid_ref, iid_ref, umf_ref, imf_ref, uneu_ref, ineu_ref,
              w0a_ref, w0b_ref, b0_ref, w1t_ref, b1_ref, out_ref):
    def pick(rows, ids):
        q0 = rows[:, 0 * HD:1 * HD]
        q1 = rows[:, 1 * HD:2 * HD]
        q2 = rows[:, 2 * HD:3 * HD]
        q3 = rows[:, 3 * HD:4 * HD]
        qq = lax.shift_right_logical(ids, QSH)[:, :HD]
        m0 = (qq & 1) == 1
        m1 = (qq & 2) == 2
        return _unpack(jnp.where(m1, jnp.where(m0, q3, q2),
                                 jnp.where(m0, q1, q0)))

    ub = uid_ref[...]
    ib = iid_ref[...]
    umf = pick(umf_ref[...], ub)
    imf = pick(imf_ref[...], ib)
    uneu = pick(uneu_ref[...], ub)
    ineu = pick(ineu_ref[...], ib)
    h0 = jax.nn.sigmoid(
        jnp.dot(uneu, w0a_ref[...], preferred_element_type=jnp.float32)
        + jnp.dot(ineu, w0b_ref[...], preferred_element_type=jnp.float32)
        + b0_ref[...])
    h1 = jax.nn.sigmoid(
        jnp.dot(h0, w1t_ref[...], preferred_element_type=jnp.float32)
        + b1_ref[...])
    gmf = jnp.sum(umf * imf, axis=1)
    out_ref[...] = gmf + jnp.sum(h1, axis=1)


_BLK = 2048


def _mlp(uidb, iidb, umf, imf, uneu, ineu, w0a, w0b, b0, w1t, b1):
    grid = (BATCH // _BLK,)
    rows_spec = pl.BlockSpec((_BLK, 128), lambda i: (i, 0))
    full = lambda shape: pl.BlockSpec(shape, lambda i: (0,) * len(shape))
    return pl.pallas_call(
        _mlp_body,
        grid=grid,
        in_specs=[
            rows_spec, rows_spec,
            rows_spec, rows_spec, rows_spec, rows_spec,
            full((D, 128)), full((D, 128)), full((1, 128)),
            full((128, 64)), full((1, 64)),
        ],
        out_specs=pl.BlockSpec((_BLK,), lambda i: (i,)),
        out_shape=jax.ShapeDtypeStruct((BATCH,), jnp.float32),
    )(uidb, iidb, umf, imf, uneu, ineu, w0a, w0b, b0, w1t, b1)


def kernel(user_id, item_id, users_mf, items_mf, users_neu, items_neu,
           W0, b0, W1, b1):
    uid = user_id.astype(jnp.int32)
    iid = item_id.astype(jnp.int32)
    eye = jnp.eye(D, dtype=jnp.float32)
    tabs = _prep([t.T for t in (users_mf, items_mf, users_neu, items_neu)],
                 eye)
    rows = _gather4(uid, iid, *tabs)
    uidb = jnp.broadcast_to(uid[:, None], (BATCH, 128))
    iidb = jnp.broadcast_to(iid[:, None], (BATCH, 128))
    w0a = W0[:, :D].T
    w0b = W0[:, D:].T
    w1t = W1.T
    return _mlp(uidb, iidb, *rows, w0a, w0b,
                b0.reshape(1, -1), w1t, b1.reshape(1, -1))


# speedup vs baseline: 3.6643x; 1.7332x over previous
import functools

import jax
import jax.numpy as jnp
from jax import lax
from jax.experimental import pallas as pl
from jax.experimental.pallas import tpu as pltpu
from jax.experimental.pallas import tpu_sc as plsc

BATCH = 16384
D = 64
N = 1000000
HSH = 19
NROW = 1 << (HSH - 1)
NC, NS = 2, 16
NW = NC * NS
B_PER_W = BATCH // NW
HALF = B_PER_W // 2
L = 16


_PBLK = 2048
_PGRID = (1 << HSH) // _PBLK
_NINB = (N + _PBLK - 1) // _PBLK


def _prep_body(*refs):
    in_refs = refs[:8]
    eye_ref = refs[8]
    out_refs = refs[9:13]
    eye = eye_ref[...]
    for t in range(4):
        halves = []
        for h in range(2):
            x = in_refs[2 * t + h][...]
            xt = lax.dot_general(x, eye, (((0,), (0,)), ((), ())),
                                 preferred_element_type=jnp.float32)
            halves.append(pltpu.bitcast(xt.astype(jnp.bfloat16), jnp.int32))
        out_refs[t][...] = jnp.concatenate(halves, axis=1)


def _prep(tabs_t, eye):
    def in_spec(h):
        base = h * _PGRID
        return pl.BlockSpec(
            (D, _PBLK), lambda i, b=base: (0, jnp.minimum(i + b, _NINB - 1)))

    in_specs = [in_spec(h) for _ in range(4) for h in range(2)]
    in_specs.append(pl.BlockSpec((D, D), lambda i: (0, 0)))
    out_spec = pl.BlockSpec((_PBLK // 2, 128), lambda i: (i, 0))
    out_t = jax.ShapeDtypeStruct((NROW, 128), jnp.int32)
    ins = []
    for t in tabs_t:
        ins.extend([t, t])
    ins.append(eye)
    return pl.pallas_call(
        _prep_body,
        grid=(_PGRID,),
        in_specs=in_specs,
        out_specs=(out_spec,) * 4,
        out_shape=(out_t,) * 4,
        compiler_params=pltpu.CompilerParams(
            fuse_transposed_lhs_in_matmul=True),
    )(*ins)


_SC_MESH = plsc.VectorSubcoreMesh(core_axis_name="c", subcore_axis_name="s")

_ROWS_T = jax.ShapeDtypeStruct((BATCH, 128), jnp.int32)


@functools.partial(
    pl.kernel,
    mesh=_SC_MESH,
    out_type=(_ROWS_T, _ROWS_T, _ROWS_T, _ROWS_T),
    scratch_types=[
        pltpu.VMEM((B_PER_W,), jnp.int32),
        pltpu.VMEM((B_PER_W,), jnp.int32),
        pltpu.VMEM((HALF, 128), jnp.int32),
        pltpu.VMEM((HALF, 128), jnp.int32),
        pltpu.SemaphoreType.DMA,
        pltpu.SemaphoreType.DMA,
        pltpu.SemaphoreType.DMA,
        pltpu.SemaphoreType.DMA,
    ],
)
def _gather4(uid_hbm, iid_hbm, umf_hbm, imf_hbm, uneu_hbm, ineu_hbm,
             out_umf, out_imf, out_uneu, out_ineu,
             idx_u, idx_i, buf_a, buf_b, sem_a, sem_b, sem_wa, sem_wb):
    wid = lax.axis_index("s") * NC + lax.axis_index("c")
    base = wid * B_PER_W
    pltpu.sync_copy(uid_hbm.at[pl.ds(base, B_PER_W)], idx_u)
    pltpu.sync_copy(iid_hbm.at[pl.ds(base, B_PER_W)], idx_i)
    mask = jnp.int32(NROW - 1)
    for k in range(B_PER_W // L):
        sl = pl.ds(k * L, L)
        idx_u[sl] = lax.shift_right_logical(idx_u[sl], 1) & mask
        idx_i[sl] = lax.shift_right_logical(idx_i[sl], 1) & mask

    jobs = ((umf_hbm, idx_u, out_umf), (imf_hbm, idx_i, out_imf),
            (uneu_hbm, idx_u, out_uneu), (ineu_hbm, idx_i, out_ineu))
    for tbl, idx, out in jobs:
        g0 = pltpu.async_copy(tbl.at[idx.at[pl.ds(0, HALF)]], buf_a, sem_a)
        g1 = pltpu.async_copy(tbl.at[idx.at[pl.ds(HALF, HALF)]], buf_b, sem_b)
        g0.wait()
        w0 = pltpu.async_copy(buf_a, out.at[pl.ds(base, HALF)], sem_wa)
        g1.wait()
        w1 = pltpu.async_copy(buf_b, out.at[pl.ds(base + HALF, HALF)], sem_wb)
        w0.wait()
        w1.wait()


def _mlp_body(uid_ref, iid_ref, umf_ref, imf_ref, uneu_ref, ineu_ref,
              w0a_ref, w0b_ref, b0_ref, w1t_ref, b1_ref, out_ref):
    def pick(rows, ids):
        idb = ids[:, :D]
        mh = lax.shift_right_logical(idb, HSH) & 1 == 1
        half = jnp.where(mh, rows[:, D:], rows[:, :D])
        modd = (idb & 1) == 1
        w = jnp.where(modd, half & jnp.int32(-65536), lax.shift_left(half, 16))
        return lax.bitcast_convert_type(w, jnp.float32)

    ub = uid_ref[...]
    ib = iid_ref[...]
    umf = pick(umf_ref[...], ub)
    imf = pick(imf_ref[...], ib)
    uneu = pick(uneu_ref[...], ub)
    ineu = pick(ineu_ref[...], ib)
    h0 = jax.nn.sigmoid(
        jnp.dot(uneu, w0a_ref[...], preferred_element_type=jnp.float32)
        + jnp.dot(ineu, w0b_ref[...], preferred_element_type=jnp.float32)
        + b0_ref[...])
    h1 = jax.nn.sigmoid(
        jnp.dot(h0, w1t_ref[...], preferred_element_type=jnp.float32)
        + b1_ref[...])
    gmf = jnp.sum(umf * imf, axis=1)
    out_ref[...] = gmf + jnp.sum(h1, axis=1)


_BLK = 2048


def _mlp(uidb, iidb, umf, imf, uneu, ineu, w0a, w0b, b0, w1t, b1):
    grid = (BATCH // _BLK,)
    rows_spec = pl.BlockSpec((_BLK, 128), lambda i: (i, 0))
    full = lambda shape: pl.BlockSpec(shape, lambda i: (0,) * len(shape))
    return pl.pallas_call(
        _mlp_body,
        grid=grid,
        in_specs=[
            rows_spec, rows_spec,
            rows_spec, rows_spec, rows_spec, rows_spec,
            full((D, 128)), full((D, 128)), full((1, 128)),
            full((128, 64)), full((1, 64)),
        ],
        out_specs=pl.BlockSpec((_BLK,), lambda i: (i,)),
        out_shape=jax.ShapeDtypeStruct((BATCH,), jnp.float32),
    )(uidb, iidb, umf, imf, uneu, ineu, w0a, w0b, b0, w1t, b1)


def kernel(user_id, item_id, users_mf, items_mf, users_neu, items_neu,
           W0, b0, W1, b1):
    uid = user_id.astype(jnp.int32)
    iid = item_id.astype(jnp.int32)
    eye = jnp.eye(D, dtype=jnp.float32)
    tabs = _prep([t.T for t in (users_mf, items_mf, users_neu, items_neu)],
                 eye)
    rows = _gather4(uid, iid, *tabs)
    uidb = jnp.broadcast_to(uid[:, None], (BATCH, 128))
    iidb = jnp.broadcast_to(iid[:, None], (BATCH, 128))
    w0a = W0[:, :D].T
    w0b = W0[:, D:].T
    w1t = W1.T
    return _mlp(uidb, iidb, *rows, w0a, w0b,
                b0.reshape(1, -1), w1t, b1.reshape(1, -1))


# speedup vs baseline: 4.3988x; 1.2005x over previous
import functools

import jax
import jax.numpy as jnp
from jax import lax
from jax.experimental import pallas as pl
from jax.experimental.pallas import tpu as pltpu
from jax.experimental.pallas import tpu_sc as plsc

BATCH = 16384
D = 64
N = 1000000
HSH = 19
NROW = 1 << (HSH - 1)
NC, NS = 2, 16
NW = NC * NS
B_PER_W = BATCH // NW
HALF = B_PER_W // 2
L = 16


_PBLK = 2048
_PGRID = (1 << HSH) // _PBLK
_NINB = (N + _PBLK - 1) // _PBLK


def _prep_body(*refs):
    in_refs = refs[:8]
    out_refs = refs[9:13]
    for t in range(4):
        halves = []
        for h in range(2):
            x = in_refs[2 * t + h][...]
            xt = jnp.swapaxes(x.astype(jnp.bfloat16), 0, 1)
            halves.append(pltpu.bitcast(xt, jnp.int32))
        out_refs[t][...] = jnp.concatenate(halves, axis=1)


def _prep(tabs_t, eye):
    def in_spec(h):
        base = h * _PGRID
        return pl.BlockSpec(
            (D, _PBLK), lambda i, b=base: (0, jnp.minimum(i + b, _NINB - 1)))

    in_specs = [in_spec(h) for _ in range(4) for h in range(2)]
    in_specs.append(pl.BlockSpec((D, D), lambda i: (0, 0)))
    out_spec = pl.BlockSpec((_PBLK // 2, 128), lambda i: (i, 0))
    out_t = jax.ShapeDtypeStruct((NROW, 128), jnp.int32)
    ins = []
    for t in tabs_t:
        ins.extend([t, t])
    ins.append(eye)
    return pl.pallas_call(
        _prep_body,
        grid=(_PGRID,),
        in_specs=in_specs,
        out_specs=(out_spec,) * 4,
        out_shape=(out_t,) * 4,
        compiler_params=pltpu.CompilerParams(
            fuse_transposed_lhs_in_matmul=True),
    )(*ins)


_SC_MESH = plsc.VectorSubcoreMesh(core_axis_name="c", subcore_axis_name="s")

_ROWS_T = jax.ShapeDtypeStruct((BATCH, 128), jnp.int32)


@functools.partial(
    pl.kernel,
    mesh=_SC_MESH,
    out_type=(_ROWS_T, _ROWS_T, _ROWS_T, _ROWS_T),
    scratch_types=[
        pltpu.VMEM((B_PER_W,), jnp.int32),
        pltpu.VMEM((B_PER_W,), jnp.int32),
        pltpu.VMEM((HALF, 128), jnp.int32),
        pltpu.VMEM((HALF, 128), jnp.int32),
        pltpu.SemaphoreType.DMA,
        pltpu.SemaphoreType.DMA,
        pltpu.SemaphoreType.DMA,
        pltpu.SemaphoreType.DMA,
    ],
)
def _gather4(uid_hbm, iid_hbm, umf_hbm, imf_hbm, uneu_hbm, ineu_hbm,
             out_umf, out_imf, out_uneu, out_ineu,
             idx_u, idx_i, buf_a, buf_b, sem_a, sem_b, sem_wa, sem_wb):
    wid = lax.axis_index("s") * NC + lax.axis_index("c")
    base = wid * B_PER_W
    pltpu.sync_copy(uid_hbm.at[pl.ds(base, B_PER_W)], idx_u)
    pltpu.sync_copy(iid_hbm.at[pl.ds(base, B_PER_W)], idx_i)
    mask = jnp.int32(NROW - 1)
    for k in range(B_PER_W // L):
        sl = pl.ds(k * L, L)
        idx_u[sl] = lax.shift_right_logical(idx_u[sl], 1) & mask
        idx_i[sl] = lax.shift_right_logical(idx_i[sl], 1) & mask

    jobs = ((umf_hbm, idx_u, out_umf), (imf_hbm, idx_i, out_imf),
            (uneu_hbm, idx_u, out_uneu), (ineu_hbm, idx_i, out_ineu))
    for tbl, idx, out in jobs:
        g0 = pltpu.async_copy(tbl.at[idx.at[pl.ds(0, HALF)]], buf_a, sem_a)
        g1 = pltpu.async_copy(tbl.at[idx.at[pl.ds(HALF, HALF)]], buf_b, sem_b)
        g0.wait()
        w0 = pltpu.async_copy(buf_a, out.at[pl.ds(base, HALF)], sem_wa)
        g1.wait()
        w1 = pltpu.async_copy(buf_b, out.at[pl.ds(base + HALF, HALF)], sem_wb)
        w0.wait()
        w1.wait()


def _mlp_body(uid_ref, iid_ref, umf_ref, imf_ref, uneu_ref, ineu_ref,
              w0a_ref, w0b_ref, b0_ref, w1t_ref, b1_ref, out_ref):
    def pick(rows, ids):
        idb = ids[:, :D]
        mh = lax.shift_right_logical(idb, HSH) & 1 == 1
        half = jnp.where(mh, rows[:, D:], rows[:, :D])
        modd = (idb & 1) == 1
        w = jnp.where(modd, half & jnp.int32(-65536), lax.shift_left(half, 16))
        return lax.bitcast_convert_type(w, jnp.float32)

    ub = uid_ref[...]
    ib = iid_ref[...]
    umf = pick(umf_ref[...], ub)
    imf = pick(imf_ref[...], ib)
    uneu = pick(uneu_ref[...], ub)
    ineu = pick(ineu_ref[...], ib)
    h0 = jax.nn.sigmoid(
        jnp.dot(uneu, w0a_ref[...], preferred_element_type=jnp.float32)
        + jnp.dot(ineu, w0b_ref[...], preferred_element_type=jnp.float32)
        + b0_ref[...])
    h1 = jax.nn.sigmoid(
        jnp.dot(h0, w1t_ref[...], preferred_element_type=jnp.float32)
        + b1_ref[...])
    gmf = jnp.sum(umf * imf, axis=1)
    out_ref[...] = gmf + jnp.sum(h1, axis=1)


_BLK = 2048


def _mlp(uidb, iidb, umf, imf, uneu, ineu, w0a, w0b, b0, w1t, b1):
    grid = (BATCH // _BLK,)
    rows_spec = pl.BlockSpec((_BLK, 128), lambda i: (i, 0))
    full = lambda shape: pl.BlockSpec(shape, lambda i: (0,) * len(shape))
    return pl.pallas_call(
        _mlp_body,
        grid=grid,
        in_specs=[
            rows_spec, rows_spec,
            rows_spec, rows_spec, rows_spec, rows_spec,
            full((D, 128)), full((D, 128)), full((1, 128)),
            full((128, 64)), full((1, 64)),
        ],
        out_specs=pl.BlockSpec((_BLK,), lambda i: (i,)),
        out_shape=jax.ShapeDtypeStruct((BATCH,), jnp.float32),
    )(uidb, iidb, umf, imf, uneu, ineu, w0a, w0b, b0, w1t, b1)


def kernel(user_id, item_id, users_mf, items_mf, users_neu, items_neu,
           W0, b0, W1, b1):
    uid = user_id.astype(jnp.int32)
    iid = item_id.astype(jnp.int32)
    eye = jnp.eye(D, dtype=jnp.float32)
    tabs = _prep([t.T for t in (users_mf, items_mf, users_neu, items_neu)],
                 eye)
    rows = _gather4(uid, iid, *tabs)
    uidb = jnp.broadcast_to(uid[:, None], (BATCH, 128))
    iidb = jnp.broadcast_to(iid[:, None], (BATCH, 128))
    w0a = W0[:, :D].T
    w0b = W0[:, D:].T
    w1t = W1.T
    return _mlp(uidb, iidb, *rows, w0a, w0b,
                b0.reshape(1, -1), w1t, b1.reshape(1, -1))


# speedup vs baseline: 4.8766x; 1.1086x over previous
import functools

import jax
import jax.numpy as jnp
from jax import lax
from jax.experimental import pallas as pl
from jax.experimental.pallas import tpu as pltpu
from jax.experimental.pallas import tpu_sc as plsc

BATCH = 16384
D = 64
N = 1000000
HSH = 19
NROW = 1 << (HSH - 1)
NC, NS = 2, 16
NW = NC * NS
B_PER_W = BATCH // NW
HALF = B_PER_W // 2
L = 16


_PBLK = 4096
_PGRID = (1 << HSH) // _PBLK
_NINB = (N + _PBLK - 1) // _PBLK


def _prep_body(*refs):
    in_refs = refs[:8]
    out_refs = refs[8:12]
    for t in range(4):
        halves = []
        for h in range(2):
            x = in_refs[2 * t + h][...]
            xt = jnp.swapaxes(x.astype(jnp.bfloat16), 0, 1)
            halves.append(pltpu.bitcast(xt, jnp.int32))
        out_refs[t][...] = jnp.concatenate(halves, axis=1)


def _prep(tabs_t):
    def in_spec(h):
        base = h * _PGRID
        return pl.BlockSpec(
            (D, _PBLK), lambda i, b=base: (0, jnp.minimum(i + b, _NINB - 1)))

    in_specs = [in_spec(h) for _ in range(4) for h in range(2)]
    out_spec = pl.BlockSpec((_PBLK // 2, 128), lambda i: (i, 0))
    out_t = jax.ShapeDtypeStruct((NROW, 128), jnp.int32)
    ins = []
    for t in tabs_t:
        ins.extend([t, t])
    return pl.pallas_call(
        _prep_body,
        grid=(_PGRID,),
        in_specs=in_specs,
        out_specs=(out_spec,) * 4,
        out_shape=(out_t,) * 4,
        compiler_params=pltpu.CompilerParams(
            dimension_semantics=("arbitrary",)),
    )(*ins)


_SC_MESH = plsc.VectorSubcoreMesh(core_axis_name="c", subcore_axis_name="s")

_ROWS_T = jax.ShapeDtypeStruct((BATCH, 128), jnp.int32)


@functools.partial(
    pl.kernel,
    mesh=_SC_MESH,
    out_type=(_ROWS_T, _ROWS_T, _ROWS_T, _ROWS_T),
    scratch_types=[
        pltpu.VMEM((B_PER_W,), jnp.int32),
        pltpu.VMEM((B_PER_W,), jnp.int32),
        pltpu.VMEM((HALF, 128), jnp.int32),
        pltpu.VMEM((HALF, 128), jnp.int32),
        pltpu.SemaphoreType.DMA,
        pltpu.SemaphoreType.DMA,
        pltpu.SemaphoreType.DMA,
        pltpu.SemaphoreType.DMA,
    ],
)
def _gather4(uid_hbm, iid_hbm, umf_hbm, imf_hbm, uneu_hbm, ineu_hbm,
             out_umf, out_imf, out_uneu, out_ineu,
             idx_u, idx_i, buf_a, buf_b, sem_a, sem_b, sem_wa, sem_wb):
    wid = lax.axis_index("s") * NC + lax.axis_index("c")
    base = wid * B_PER_W
    pltpu.sync_copy(uid_hbm.at[pl.ds(base, B_PER_W)], idx_u)
    pltpu.sync_copy(iid_hbm.at[pl.ds(base, B_PER_W)], idx_i)
    mask = jnp.int32(NROW - 1)
    for k in range(B_PER_W // L):
        sl = pl.ds(k * L, L)
        idx_u[sl] = lax.shift_right_logical(idx_u[sl], 1) & mask
        idx_i[sl] = lax.shift_right_logical(idx_i[sl], 1) & mask

    jobs = ((umf_hbm, idx_u, out_umf), (imf_hbm, idx_i, out_imf),
            (uneu_hbm, idx_u, out_uneu), (ineu_hbm, idx_i, out_ineu))
    for tbl, idx, out in jobs:
        g0 = pltpu.async_copy(tbl.at[idx.at[pl.ds(0, HALF)]], buf_a, sem_a)
        g1 = pltpu.async_copy(tbl.at[idx.at[pl.ds(HALF, HALF)]], buf_b, sem_b)
        g0.wait()
        w0 = pltpu.async_copy(buf_a, out.at[pl.ds(base, HALF)], sem_wa)
        g1.wait()
        w1 = pltpu.async_copy(buf_b, out.at[pl.ds(base + HALF, HALF)], sem_wb)
        w0.wait()
        w1.wait()


def _mlp_body(uid_ref, iid_ref, umf_ref, imf_ref, uneu_ref, ineu_ref,
              w0a_ref, w0b_ref, b0_ref, w1t_ref, b1_ref, out_ref):
    def pick(rows, ids):
        idb = ids[:, :D]
        mh = lax.shift_right_logical(idb, HSH) & 1 == 1
        half = jnp.where(mh, rows[:, D:], rows[:, :D])
        modd = (idb & 1) == 1
        w = jnp.where(modd, half & jnp.int32(-65536), lax.shift_left(half, 16))
        return lax.bitcast_convert_type(w, jnp.float32)

    ub = uid_ref[...]
    ib = iid_ref[...]
    umf = pick(umf_ref[...], ub)
    imf = pick(imf_ref[...], ib)
    uneu = pick(uneu_ref[...], ub)
    ineu = pick(ineu_ref[...], ib)
    h0 = jax.nn.sigmoid(
        jnp.dot(uneu, w0a_ref[...], preferred_element_type=jnp.float32)
        + jnp.dot(ineu, w0b_ref[...], preferred_element_type=jnp.float32)
        + b0_ref[...])
    h1 = jax.nn.sigmoid(
        jnp.dot(h0, w1t_ref[...], preferred_element_type=jnp.float32)
        + b1_ref[...])
    gmf = jnp.sum(umf * imf, axis=1)
    out_ref[...] = gmf + jnp.sum(h1, axis=1)


_BLK = 2048


def _mlp(uidb, iidb, umf, imf, uneu, ineu, w0a, w0b, b0, w1t, b1):
    grid = (BATCH // _BLK,)
    rows_spec = pl.BlockSpec((_BLK, 128), lambda i: (i, 0))
    full = lambda shape: pl.BlockSpec(shape, lambda i: (0,) * len(shape))
    return pl.pallas_call(
        _mlp_body,
        grid=grid,
        in_specs=[
            rows_spec, rows_spec,
            rows_spec, rows_spec, rows_spec, rows_spec,
            full((D, 128)), full((D, 128)), full((1, 128)),
            full((128, 64)), full((1, 64)),
        ],
        out_specs=pl.BlockSpec((_BLK,), lambda i: (i,)),
        out_shape=jax.ShapeDtypeStruct((BATCH,), jnp.float32),
    )(uidb, iidb, umf, imf, uneu, ineu, w0a, w0b, b0, w1t, b1)


def kernel(user_id, item_id, users_mf, items_mf, users_neu, items_neu,
           W0, b0, W1, b1):
    uid = user_id.astype(jnp.int32)
    iid = item_id.astype(jnp.int32)
    tabs = _prep([t.T for t in (users_mf, items_mf, users_neu, items_neu)])
    rows = _gather4(uid, iid, *tabs)
    uidb = jnp.broadcast_to(uid[:, None], (BATCH, 128))
    iidb = jnp.broadcast_to(iid[:, None], (BATCH, 128))
    w0a = W0[:, :D].T
    w0b = W0[:, D:].T
    w1t = W1.T
    return _mlp(uidb, iidb, *rows, w0a, w0b,
                b0.reshape(1, -1), w1t, b1.reshape(1, -1))


# speedup vs baseline: 4.9672x; 1.0186x over previous
import functools

import jax
import jax.numpy as jnp
from jax import lax
from jax.experimental import pallas as pl
from jax.experimental.pallas import tpu as pltpu
from jax.experimental.pallas import tpu_sc as plsc

BATCH = 16384
D = 64
N = 1000000
HSH = 19
NROW = 1 << (HSH - 1)
NC, NS = 2, 16
NW = NC * NS
B_PER_W = BATCH // NW
HALF = B_PER_W // 2
L = 16


_PBLK = 8192
_PGRID = (1 << HSH) // _PBLK
_NINB = (N + _PBLK - 1) // _PBLK


def _prep_body(*refs):
    in_refs = refs[:8]
    out_refs = refs[8:12]
    for t in range(4):
        halves = []
        for h in range(2):
            x = in_refs[2 * t + h][...]
            xt = jnp.swapaxes(x.astype(jnp.bfloat16), 0, 1)
            halves.append(pltpu.bitcast(xt, jnp.int32))
        out_refs[t][...] = jnp.concatenate(halves, axis=1)


def _prep(tabs_t):
    def in_spec(h):
        base = h * _PGRID
        return pl.BlockSpec(
            (D, _PBLK), lambda i, b=base: (0, jnp.minimum(i + b, _NINB - 1)))

    in_specs = [in_spec(h) for _ in range(4) for h in range(2)]
    out_spec = pl.BlockSpec((_PBLK // 2, 128), lambda i: (i, 0))
    out_t = jax.ShapeDtypeStruct((NROW, 128), jnp.int32)
    ins = []
    for t in tabs_t:
        ins.extend([t, t])
    return pl.pallas_call(
        _prep_body,
        grid=(_PGRID,),
        in_specs=in_specs,
        out_specs=(out_spec,) * 4,
        out_shape=(out_t,) * 4,
        compiler_params=pltpu.CompilerParams(
            dimension_semantics=("arbitrary",),
            vmem_limit_bytes=100 * 1024 * 1024),
    )(*ins)


_SC_MESH = plsc.VectorSubcoreMesh(core_axis_name="c", subcore_axis_name="s")

_ROWS_T = jax.ShapeDtypeStruct((BATCH, 128), jnp.int32)


@functools.partial(
    pl.kernel,
    mesh=_SC_MESH,
    out_type=(_ROWS_T, _ROWS_T, _ROWS_T, _ROWS_T),
    scratch_types=[
        pltpu.VMEM((B_PER_W,), jnp.int32),
        pltpu.VMEM((B_PER_W,), jnp.int32),
        pltpu.VMEM((HALF, 128), jnp.int32),
        pltpu.VMEM((HALF, 128), jnp.int32),
        pltpu.SemaphoreType.DMA,
        pltpu.SemaphoreType.DMA,
        pltpu.SemaphoreType.DMA,
        pltpu.SemaphoreType.DMA,
    ],
)
def _gather4(uid_hbm, iid_hbm, umf_hbm, imf_hbm, uneu_hbm, ineu_hbm,
             out_umf, out_imf, out_uneu, out_ineu,
             idx_u, idx_i, buf_a, buf_b, sem_a, sem_b, sem_wa, sem_wb):
    wid = lax.axis_index("s") * NC + lax.axis_index("c")
    base = wid * B_PER_W
    pltpu.sync_copy(uid_hbm.at[pl.ds(base, B_PER_W)], idx_u)
    pltpu.sync_copy(iid_hbm.at[pl.ds(base, B_PER_W)], idx_i)
    mask = jnp.int32(NROW - 1)
    for k in range(B_PER_W // L):
        sl = pl.ds(k * L, L)
        idx_u[sl] = lax.shift_right_logical(idx_u[sl], 1) & mask
        idx_i[sl] = lax.shift_right_logical(idx_i[sl], 1) & mask

    jobs = ((umf_hbm, idx_u, out_umf), (imf_hbm, idx_i, out_imf),
            (uneu_hbm, idx_u, out_uneu), (ineu_hbm, idx_i, out_ineu))
    for tbl, idx, out in jobs:
        g0 = pltpu.async_copy(tbl.at[idx.at[pl.ds(0, HALF)]], buf_a, sem_a)
        g1 = pltpu.async_copy(tbl.at[idx.at[pl.ds(HALF, HALF)]], buf_b, sem_b)
        g0.wait()
        w0 = pltpu.async_copy(buf_a, out.at[pl.ds(base, HALF)], sem_wa)
        g1.wait()
        w1 = pltpu.async_copy(buf_b, out.at[pl.ds(base + HALF, HALF)], sem_wb)
        w0.wait()
        w1.wait()


def _mlp_body(uid_ref, iid_ref, umf_ref, imf_ref, uneu_ref, ineu_ref,
              w0a_ref, w0b_ref, b0_ref, w1t_ref, b1_ref, out_ref):
    def pick(rows, ids):
        idb = ids[:, :D]
        mh = lax.shift_right_logical(idb, HSH) & 1 == 1
        half = jnp.where(mh, rows[:, D:], rows[:, :D])
        modd = (idb & 1) == 1
        w = jnp.where(modd, half & jnp.int32(-65536), lax.shift_left(half, 16))
        return lax.bitcast_convert_type(w, jnp.float32)

    ub = uid_ref[...]
    ib = iid_ref[...]
    umf = pick(umf_ref[...], ub)
    imf = pick(imf_ref[...], ib)
    uneu = pick(uneu_ref[...], ub)
    ineu = pick(ineu_ref[...], ib)
    h0 = jax.nn.sigmoid(
        jnp.dot(uneu, w0a_ref[...], preferred_element_type=jnp.float32)
        + jnp.dot(ineu, w0b_ref[...], preferred_element_type=jnp.float32)
        + b0_ref[...])
    h1 = jax.nn.sigmoid(
        jnp.dot(h0, w1t_ref[...], preferred_element_type=jnp.float32)
        + b1_ref[...])
    gmf = jnp.sum(umf * imf, axis=1)
    out_ref[...] = gmf + jnp.sum(h1, axis=1)


_BLK = 2048


def _mlp(uidb, iidb, umf, imf, uneu, ineu, w0a, w0b, b0, w1t, b1):
    grid = (BATCH // _BLK,)
    rows_spec = pl.BlockSpec((_BLK, 128), lambda i: (i, 0))
    full = lambda shape: pl.BlockSpec(shape, lambda i: (0,) * len(shape))
    return pl.pallas_call(
        _mlp_body,
        grid=grid,
        in_specs=[
            rows_spec, rows_spec,
            rows_spec, rows_spec, rows_spec, rows_spec,
            full((D, 128)), full((D, 128)), full((1, 128)),
            full((128, 64)), full((1, 64)),
        ],
        out_specs=pl.BlockSpec((_BLK,), lambda i: (i,)),
        out_shape=jax.ShapeDtypeStruct((BATCH,), jnp.float32),
    )(uidb, iidb, umf, imf, uneu, ineu, w0a, w0b, b0, w1t, b1)


def kernel(user_id, item_id, users_mf, items_mf, users_neu, items_neu,
           W0, b0, W1, b1):
    uid = user_id.astype(jnp.int32)
    iid = item_id.astype(jnp.int32)
    tabs = _prep([t.T for t in (users_mf, items_mf, users_neu, items_neu)])
    rows = _gather4(uid, iid, *tabs)
    uidb = jnp.broadcast_to(uid[:, None], (BATCH, 128))
    iidb = jnp.broadcast_to(iid[:, None], (BATCH, 128))
    w0a = W0[:, :D].T
    w0b = W0[:, D:].T
    w1t = W1.T
    return _mlp(uidb, iidb, *rows, w0a, w0b,
                b0.reshape(1, -1), w1t, b1.reshape(1, -1))


# speedup vs baseline: 4.9921x; 1.0050x over previous
import functools

import jax
import jax.numpy as jnp
from jax import lax
from jax.experimental import pallas as pl
from jax.experimental.pallas import tpu as pltpu
from jax.experimental.pallas import tpu_sc as plsc

BATCH = 16384
D = 64
N = 1000000
HSH = 19
NROW = 1 << (HSH - 1)
NC, NS = 2, 16
NW = NC * NS
B_PER_W = BATCH // NW
HALF = B_PER_W // 2
L = 16


_PBLK = 8192
_PGRID = (1 << HSH) // _PBLK
_NINB = (N + _PBLK - 1) // _PBLK


def _prep_body(*refs):
    in_refs = refs[:8]
    out_refs = refs[8:12]
    for t in range(4):
        halves = []
        for h in range(2):
            x = in_refs[2 * t + h][...]
            if x.dtype != jnp.bfloat16:
                x = x.astype(jnp.bfloat16)
            xt = jnp.swapaxes(x, 0, 1)
            halves.append(pltpu.bitcast(xt, jnp.int32))
        out_refs[t][...] = jnp.concatenate(halves, axis=1)


def _prep(tabs_t):
    def in_spec(h):
        base = h * _PGRID
        return pl.BlockSpec(
            (D, _PBLK), lambda i, b=base: (0, jnp.minimum(i + b, _NINB - 1)))

    in_specs = [in_spec(h) for _ in range(4) for h in range(2)]
    out_spec = pl.BlockSpec((_PBLK // 2, 128), lambda i: (i, 0))
    out_t = jax.ShapeDtypeStruct((NROW, 128), jnp.int32)
    ins = []
    for t in tabs_t:
        ins.extend([t, t])
    return pl.pallas_call(
        _prep_body,
        grid=(_PGRID,),
        in_specs=in_specs,
        out_specs=(out_spec,) * 4,
        out_shape=(out_t,) * 4,
        compiler_params=pltpu.CompilerParams(
            dimension_semantics=("arbitrary",),
            vmem_limit_bytes=100 * 1024 * 1024),
    )(*ins)


_SC_MESH = plsc.VectorSubcoreMesh(core_axis_name="c", subcore_axis_name="s")

_ROWS_T = jax.ShapeDtypeStruct((BATCH, 128), jnp.int32)


@functools.partial(
    pl.kernel,
    mesh=_SC_MESH,
    out_type=(_ROWS_T, _ROWS_T, _ROWS_T, _ROWS_T),
    scratch_types=[
        pltpu.VMEM((B_PER_W,), jnp.int32),
        pltpu.VMEM((B_PER_W,), jnp.int32),
        pltpu.VMEM((HALF, 128), jnp.int32),
        pltpu.VMEM((HALF, 128), jnp.int32),
        pltpu.SemaphoreType.DMA,
        pltpu.SemaphoreType.DMA,
        pltpu.SemaphoreType.DMA,
        pltpu.SemaphoreType.DMA,
    ],
)
def _gather4(uid_hbm, iid_hbm, umf_hbm, imf_hbm, uneu_hbm, ineu_hbm,
             out_umf, out_imf, out_uneu, out_ineu,
             idx_u, idx_i, buf_a, buf_b, sem_a, sem_b, sem_wa, sem_wb):
    wid = lax.axis_index("s") * NC + lax.axis_index("c")
    base = wid * B_PER_W
    pltpu.sync_copy(uid_hbm.at[pl.ds(base, B_PER_W)], idx_u)
    pltpu.sync_copy(iid_hbm.at[pl.ds(base, B_PER_W)], idx_i)
    mask = jnp.int32(NROW - 1)
    for k in range(B_PER_W // L):
        sl = pl.ds(k * L, L)
        idx_u[sl] = lax.shift_right_logical(idx_u[sl], 1) & mask
        idx_i[sl] = lax.shift_right_logical(idx_i[sl], 1) & mask

    jobs = ((umf_hbm, idx_u, out_umf), (imf_hbm, idx_i, out_imf),
            (uneu_hbm, idx_u, out_uneu), (ineu_hbm, idx_i, out_ineu))
    for tbl, idx, out in jobs:
        g0 = pltpu.async_copy(tbl.at[idx.at[pl.ds(0, HALF)]], buf_a, sem_a)
        g1 = pltpu.async_copy(tbl.at[idx.at[pl.ds(HALF, HALF)]], buf_b, sem_b)
        g0.wait()
        w0 = pltpu.async_copy(buf_a, out.at[pl.ds(base, HALF)], sem_wa)
        g1.wait()
        w1 = pltpu.async_copy(buf_b, out.at[pl.ds(base + HALF, HALF)], sem_wb)
        w0.wait()
        w1.wait()


def _mlp_body(uid_ref, iid_ref, umf_ref, imf_ref, uneu_ref, ineu_ref,
              w0a_ref, w0b_ref, b0_ref, w1t_ref, b1_ref, out_ref):
    def pick(rows, ids):
        idb = ids[:, :D]
        mh = lax.shift_right_logical(idb, HSH) & 1 == 1
        half = jnp.where(mh, rows[:, D:], rows[:, :D])
        modd = (idb & 1) == 1
        w = jnp.where(modd, half & jnp.int32(-65536), lax.shift_left(half, 16))
        return lax.bitcast_convert_type(w, jnp.float32)

    ub = uid_ref[...]
    ib = iid_ref[...]
    umf = pick(umf_ref[...], ub)
    imf = pick(imf_ref[...], ib)
    uneu = pick(uneu_ref[...], ub)
    ineu = pick(ineu_ref[...], ib)
    h0 = jax.nn.sigmoid(
        jnp.dot(uneu, w0a_ref[...], preferred_element_type=jnp.float32)
        + jnp.dot(ineu, w0b_ref[...], preferred_element_type=jnp.float32)
        + b0_ref[...])
    h1 = jax.nn.sigmoid(
        jnp.dot(h0, w1t_ref[...], preferred_element_type=jnp.float32)
        + b1_ref[...])
    gmf = jnp.sum(umf * imf, axis=1)
    out_ref[...] = gmf + jnp.sum(h1, axis=1)


_BLK = 2048


def _mlp(uidb, iidb, umf, imf, uneu, ineu, w0a, w0b, b0, w1t, b1):
    grid = (BATCH // _BLK,)
    rows_spec = pl.BlockSpec((_BLK, 128), lambda i: (i, 0))
    full = lambda shape: pl.BlockSpec(shape, lambda i: (0,) * len(shape))
    return pl.pallas_call(
        _mlp_body,
        grid=grid,
        in_specs=[
            rows_spec, rows_spec,
            rows_spec, rows_spec, rows_spec, rows_spec,
            full((D, 128)), full((D, 128)), full((1, 128)),
            full((128, 64)), full((1, 64)),
        ],
        out_specs=pl.BlockSpec((_BLK,), lambda i: (i,)),
        out_shape=jax.ShapeDtypeStruct((BATCH,), jnp.float32),
    )(uidb, iidb, umf, imf, uneu, ineu, w0a, w0b, b0, w1t, b1)


def kernel(user_id, item_id, users_mf, items_mf, users_neu, items_neu,
           W0, b0, W1, b1):
    uid = user_id.astype(jnp.int32)
    iid = item_id.astype(jnp.int32)
    tabs = _prep([t.T for t in (users_mf, items_mf, users_neu, items_neu)])
    rows = _gather4(uid, iid, *tabs)
    uidb = jnp.broadcast_to(uid[:, None], (BATCH, 128))
    iidb = jnp.broadcast_to(iid[:, None], (BATCH, 128))
    w0a = W0[:, :D].T
    w0b = W0[:, D:].T
    w1t = W1.T
    return _mlp(uidb, iidb, *rows, w0a, w0b,
                b0.reshape(1, -1), w1t, b1.reshape(1, -1))
